# baseline (device time: 30102 ns/iter reference)
import jax
import jax.numpy as jnp
from jax import lax
from jax.experimental import pallas as pl
from jax.experimental.pallas import tpu as pltpu

N_DEV = 4
B = 2
SQ = 256
QROW = SQ // N_DEV
SKV = 256
HQ = 4
H_ALL = 16
DH = 64
BLK = 64
D_MODEL = 512
NEG = -1e9


def kernel(x, Wq, K_ext, V_ext, Wo):
    f32 = jnp.float32
    bf16 = jnp.bfloat16

    Kb = jnp.reshape(K_ext.astype(bf16), (B, SKV, H_ALL * DH))
    Vb = jnp.reshape(V_ext.astype(bf16), (B, SKV, H_ALL * DH))
    xb = x.astype(bf16)
    wqb = Wq.astype(bf16)
    wob = Wo.astype(bf16)

    def body(x_ref, wq_ref, kb_ref, vb_ref, wo_ref, out_ref,
             kvbuf, sbuf, rbuf, ystage, qrecv, agsend, agrecv,
             copy_sems, kv_send_sems, kv_recv_sems, r_recv_sem, fwd_send_sem,
             rs_send_sems, rs_recv_sems, ag_send_sems, ag_recv_sems):
        my = lax.axis_index("i")

        barrier = pltpu.get_barrier_semaphore()
        for d in range(N_DEV):
            pl.semaphore_signal(barrier, inc=1, device_id=(d,),
                                device_id_type=pl.DeviceIdType.MESH)
        pl.semaphore_wait(barrier, N_DEV)

        def slice_rdma(j):
            return pltpu.make_async_remote_copy(
                src_ref=sbuf.at[j - 1], dst_ref=kvbuf,
                send_sem=kv_send_sems.at[j - 1],
                recv_sem=kv_recv_sems.at[0],
                device_id=(j,), device_id_type=pl.DeviceIdType.MESH)

        def relay_in_rdma(t):
            return pltpu.make_async_remote_copy(
                src_ref=sbuf.at[1, t], dst_ref=rbuf,
                send_sem=kv_send_sems.at[1 if t == 0 else 3],
                recv_sem=r_recv_sem,
                device_id=(1 if t == 0 else 3,),
                device_id_type=pl.DeviceIdType.MESH)

        def relay_fwd_rdma(t):
            return pltpu.make_async_remote_copy(
                src_ref=rbuf, dst_ref=kvbuf.at[t],
                send_sem=fwd_send_sem, recv_sem=kv_recv_sems.at[t],
                device_id=(2,), device_id_type=pl.DeviceIdType.MESH)

        @pl.when(my == 0)
        def _():
            copies = []
            for j in range(1, N_DEV):
                lo, hi = j * HQ * DH, (j + 1) * HQ * DH
                for t, ref in ((0, kb_ref), (1, vb_ref)):
                    c = pltpu.make_async_copy(
                        ref.at[:, :, lo:hi], sbuf.at[j - 1, t],
                        copy_sems.at[2 * (j - 1) + t])
                    c.start()
                    copies.append(c)
            own = []
            for t, ref in ((0, kb_ref), (1, vb_ref)):
                c = pltpu.make_async_copy(ref.at[:, :, 0:HQ * DH],
                                          kvbuf.at[t], copy_sems.at[6 + t])
                c.start()
                own.append(c)
            copies[2].wait()
            copies[3].wait()
            relay_in_rdma(0).start()
            relay_in_rdma(1).start()
            copies[0].wait()
            copies[1].wait()
            slice_rdma(1).start()
            copies[4].wait()
            copies[5].wait()
            slice_rdma(3).start()
            own[0].wait()
            own[1].wait()

        for r, t in ((1, 0), (3, 1)):
            @pl.when(my == r)
            def _(t=t):
                relay_in_rdma(t).wait_recv()
                relay_fwd_rdma(t).start()

        q = [jnp.dot(x_ref[b], wq_ref[...], preferred_element_type=f32)
             for b in range(B)]

        @pl.when((my == 1) | (my == 3))
        def _():
            slice_rdma(1).wait_recv()

        @pl.when(my == 2)
        def _():
            relay_fwd_rdma(0).wait_recv()
            relay_fwd_rdma(1).wait_recv()

        rb = lax.broadcasted_iota(jnp.int32, (SQ, SKV), 0) // BLK
        cb = lax.broadcasted_iota(jnp.int32, (SQ, SKV), 1) // BLK
        madd = jnp.where(cb <= rb, 0.0, NEG).astype(f32)

        def rs_rdma(src_dev, dst_dev, b):
            return pltpu.make_async_remote_copy(
                src_ref=ystage.at[b, pl.ds(dst_dev * QROW, QROW)],
                dst_ref=qrecv.at[src_dev, b],
                send_sem=rs_send_sems.at[b, dst_dev],
                recv_sem=rs_recv_sems.at[b, src_dev],
                device_id=(dst_dev,), device_id_type=pl.DeviceIdType.MESH)

        def ag_rdma(src_dev, dst_dev, b):
            return pltpu.make_async_remote_copy(
                src_ref=agsend.at[b],
                dst_ref=agrecv.at[src_dev, b],
                send_sem=ag_send_sems.at[b, dst_dev],
                recv_sem=ag_recv_sems.at[b, src_dev],
                device_id=(dst_dev,), device_id_type=pl.DeviceIdType.MESH)

        for b in range(B):
            ctxs = []
            for h in range(HQ):
                k_h = kvbuf[0, b, :, h * DH:(h + 1) * DH]
                v_h = kvbuf[1, b, :, h * DH:(h + 1) * DH]
                q_h = q[b][:, h * DH:(h + 1) * DH].astype(bf16)
                s = lax.dot_general(q_h, k_h, (((1,), (1,)), ((), ())),
                                    preferred_element_type=f32)
                w = jnp.exp(s * 0.125 + madd)
                p = w / jnp.sum(w, axis=1, keepdims=True)
                ctxs.append(lax.dot_general(
                    p.astype(bf16), v_h, (((1,), (0,)), ((), ())),
                    preferred_element_type=f32))
            ctx = jnp.concatenate(ctxs, axis=1)
            y_b = jnp.dot(ctx.astype(bf16), wo_ref[...],
                          preferred_element_type=f32)
            for i in range(N_DEV):
                @pl.when(my == i)
                def _(i=i, y_b=y_b, b=b):
                    ystage[b] = y_b.astype(bf16)
                    qrecv[i, b] = ystage[b, pl.ds(i * QROW, QROW)]
                    for j in range(N_DEV):
                        if j != i:
                            rs_rdma(i, j, b).start()

        for i in range(N_DEV):
            @pl.when(my == i)
            def _(i=i):
                for b in range(B):
                    for j in range(N_DEV):
                        if j != i:
                            rs_rdma(j, i, b).wait_recv()
                    sq = (qrecv[0, b].astype(f32) + qrecv[1, b].astype(f32)
                          + qrecv[2, b].astype(f32) + qrecv[3, b].astype(f32))
                    out_ref[b, i * QROW:(i + 1) * QROW] = sq
                    agsend[b] = sq.astype(bf16)
                    for j in range(N_DEV):
                        if j != i:
                            ag_rdma(i, j, b).start()
                for b in range(B):
                    for j in range(N_DEV):
                        if j != i:
                            ag_rdma(j, i, b).wait_recv()
                            out_ref[b, j * QROW:(j + 1) * QROW] = (
                                agrecv[j, b].astype(f32))
                for b in range(B):
                    for j in range(N_DEV):
                        if j != i:
                            rs_rdma(i, j, b).wait_send()
                            ag_rdma(i, j, b).wait_send()
                if i == 0:
                    slice_rdma(1).wait_send()
                    slice_rdma(3).wait_send()
                    relay_in_rdma(0).wait_send()
                    relay_in_rdma(1).wait_send()
                if i in (1, 3):
                    relay_fwd_rdma(0 if i == 1 else 1).wait_send()

    return pl.pallas_call(
        body,
        out_shape=jax.ShapeDtypeStruct((B, SQ, D_MODEL), jnp.float32),
        in_specs=[
            pl.BlockSpec(memory_space=pltpu.VMEM),
            pl.BlockSpec(memory_space=pltpu.VMEM),
            pl.BlockSpec(memory_space=pl.ANY),
            pl.BlockSpec(memory_space=pl.ANY),
            pl.BlockSpec(memory_space=pltpu.VMEM),
        ],
        out_specs=pl.BlockSpec(memory_space=pltpu.VMEM),
        scratch_shapes=[
            pltpu.VMEM((2, B, SKV, HQ * DH), jnp.bfloat16),
            pltpu.VMEM((N_DEV - 1, 2, B, SKV, HQ * DH), jnp.bfloat16),
            pltpu.VMEM((B, SKV, HQ * DH), jnp.bfloat16),
            pltpu.VMEM((B, SQ, D_MODEL), jnp.bfloat16),
            pltpu.VMEM((N_DEV, B, QROW, D_MODEL), jnp.bfloat16),
            pltpu.VMEM((B, QROW, D_MODEL), jnp.bfloat16),
            pltpu.VMEM((N_DEV, B, QROW, D_MODEL), jnp.bfloat16),
            pltpu.SemaphoreType.DMA((8,)),
            pltpu.SemaphoreType.DMA((N_DEV,)),
            pltpu.SemaphoreType.DMA((2,)),
            pltpu.SemaphoreType.DMA,
            pltpu.SemaphoreType.DMA,
            pltpu.SemaphoreType.DMA((B, N_DEV)),
            pltpu.SemaphoreType.DMA((B, N_DEV)),
            pltpu.SemaphoreType.DMA((B, N_DEV)),
            pltpu.SemaphoreType.DMA((B, N_DEV)),
        ],
        compiler_params=pltpu.CompilerParams(collective_id=0),
    )(xb, wqb, Kb, Vb, wob)


# device time: 29631 ns/iter; 1.0159x vs baseline; 1.0159x over previous
import jax
import jax.numpy as jnp
from jax import lax
from jax.experimental import pallas as pl
from jax.experimental.pallas import tpu as pltpu

N_DEV = 4
B = 2
SQ = 256
QROW = SQ // N_DEV
SKV = 256
HQ = 4
H_ALL = 16
DH = 64
BLK = 64
D_MODEL = 512
NEG = -1e9


def kernel(x, Wq, K_ext, V_ext, Wo):
    f32 = jnp.float32
    bf16 = jnp.bfloat16

    Kb = jnp.reshape(K_ext.astype(bf16), (B, SKV, H_ALL * DH))
    Vb = jnp.reshape(V_ext.astype(bf16), (B, SKV, H_ALL * DH))
    xb = x.astype(bf16)
    wqb = Wq.astype(bf16)
    wob = Wo.astype(bf16)

    def body(x_ref, wq_ref, kb_ref, vb_ref, wo_ref, out_ref,
             kvbuf, sbuf, rbuf, ystage, qrecv, agsend, agrecv,
             copy_sems, kv_send_sems, kv_recv_sems, r_recv_sem, fwd_send_sem,
             rs_send_sems, rs_recv_sems, ag_send_sems, ag_recv_sems):
        my = lax.axis_index("i")

        barrier = pltpu.get_barrier_semaphore()
        for d in range(N_DEV):
            pl.semaphore_signal(barrier, inc=1, device_id=(d,),
                                device_id_type=pl.DeviceIdType.MESH)
        pl.semaphore_wait(barrier, N_DEV)

        def slice_rdma(j, t):
            return pltpu.make_async_remote_copy(
                src_ref=sbuf.at[j - 1, t], dst_ref=kvbuf.at[t],
                send_sem=kv_send_sems.at[(j - 1) * 2 + t],
                recv_sem=kv_recv_sems.at[t],
                device_id=(j,), device_id_type=pl.DeviceIdType.MESH)

        def relay_in_rdma(t):
            return pltpu.make_async_remote_copy(
                src_ref=sbuf.at[1, t], dst_ref=rbuf,
                send_sem=kv_send_sems.at[6 + t],
                recv_sem=r_recv_sem,
                device_id=(1 if t == 0 else 3,),
                device_id_type=pl.DeviceIdType.MESH)

        def relay_fwd_rdma(t):
            return pltpu.make_async_remote_copy(
                src_ref=rbuf, dst_ref=kvbuf.at[t],
                send_sem=fwd_send_sem, recv_sem=kv_recv_sems.at[t],
                device_id=(2,), device_id_type=pl.DeviceIdType.MESH)

        @pl.when(my == 0)
        def _():
            copies = []
            for j in range(1, N_DEV):
                lo, hi = j * HQ * DH, (j + 1) * HQ * DH
                for t, ref in ((0, kb_ref), (1, vb_ref)):
                    c = pltpu.make_async_copy(
                        ref.at[:, :, lo:hi], sbuf.at[j - 1, t],
                        copy_sems.at[2 * (j - 1) + t])
                    c.start()
                    copies.append(c)
            own = []
            for t, ref in ((0, kb_ref), (1, vb_ref)):
                c = pltpu.make_async_copy(ref.at[:, :, 0:HQ * DH],
                                          kvbuf.at[t], copy_sems.at[6 + t])
                c.start()
                own.append(c)
            copies[2].wait()
            copies[3].wait()
            relay_in_rdma(0).start()
            relay_in_rdma(1).start()
            copies[0].wait()
            copies[4].wait()
            slice_rdma(1, 0).start()
            slice_rdma(3, 0).start()
            copies[1].wait()
            copies[5].wait()
            slice_rdma(1, 1).start()
            slice_rdma(3, 1).start()
            own[0].wait()
            own[1].wait()

        for r, t in ((1, 0), (3, 1)):
            @pl.when(my == r)
            def _(t=t):
                relay_in_rdma(t).wait_recv()
                relay_fwd_rdma(t).start()

        q = [jnp.dot(x_ref[b], wq_ref[...], preferred_element_type=f32)
             for b in range(B)]

        @pl.when(my != 0)
        def _():
            slice_rdma(1, 0).wait_recv()

        rb = lax.broadcasted_iota(jnp.int32, (SQ, SKV), 0) // BLK
        cb = lax.broadcasted_iota(jnp.int32, (SQ, SKV), 1) // BLK
        madd = jnp.where(cb <= rb, 0.0, NEG).astype(f32)

        def rs_rdma(src_dev, dst_dev, b):
            return pltpu.make_async_remote_copy(
                src_ref=ystage.at[b, pl.ds(dst_dev * QROW, QROW)],
                dst_ref=qrecv.at[src_dev, b],
                send_sem=rs_send_sems.at[b, dst_dev],
                recv_sem=rs_recv_sems.at[b, src_dev],
                device_id=(dst_dev,), device_id_type=pl.DeviceIdType.MESH)

        def ag_rdma(src_dev, dst_dev, b):
            return pltpu.make_async_remote_copy(
                src_ref=agsend.at[b],
                dst_ref=agrecv.at[src_dev, b],
                send_sem=ag_send_sems.at[b, dst_dev],
                recv_sem=ag_recv_sems.at[b, src_dev],
                device_id=(dst_dev,), device_id_type=pl.DeviceIdType.MESH)

        ps = [[None] * HQ for _ in range(B)]
        for b in range(B):
            for h in range(HQ):
                k_h = kvbuf[0, b, :, h * DH:(h + 1) * DH]
                q_h = q[b][:, h * DH:(h + 1) * DH].astype(bf16)
                s = lax.dot_general(q_h, k_h, (((1,), (1,)), ((), ())),
                                    preferred_element_type=f32)
                w = jnp.exp(s * 0.125 + madd)
                ps[b][h] = (w / jnp.sum(w, axis=1, keepdims=True)).astype(bf16)

        @pl.when(my != 0)
        def _():
            slice_rdma(1, 1).wait_recv()

        for b in range(B):
            ctxs = [lax.dot_general(
                ps[b][h], kvbuf[1, b, :, h * DH:(h + 1) * DH],
                (((1,), (0,)), ((), ())), preferred_element_type=f32)
                for h in range(HQ)]
            ctx = jnp.concatenate(ctxs, axis=1)
            y_b = jnp.dot(ctx.astype(bf16), wo_ref[...],
                          preferred_element_type=f32)
            for i in range(N_DEV):
                @pl.when(my == i)
                def _(i=i, y_b=y_b, b=b):
                    ystage[b] = y_b.astype(bf16)
                    qrecv[i, b] = ystage[b, pl.ds(i * QROW, QROW)]
                    for j in range(N_DEV):
                        if j != i:
                            rs_rdma(i, j, b).start()

        for i in range(N_DEV):
            @pl.when(my == i)
            def _(i=i):
                for b in range(B):
                    for j in range(N_DEV):
                        if j != i:
                            rs_rdma(j, i, b).wait_recv()
                    sq = (qrecv[0, b].astype(f32) + qrecv[1, b].astype(f32)
                          + qrecv[2, b].astype(f32) + qrecv[3, b].astype(f32))
                    out_ref[b, i * QROW:(i + 1) * QROW] = sq
                    agsend[b] = sq.astype(bf16)
                    for j in range(N_DEV):
                        if j != i:
                            ag_rdma(i, j, b).start()
                for b in range(B):
                    for j in range(N_DEV):
                        if j != i:
                            ag_rdma(j, i, b).wait_recv()
                            out_ref[b, j * QROW:(j + 1) * QROW] = (
                                agrecv[j, b].astype(f32))
                for b in range(B):
                    for j in range(N_DEV):
                        if j != i:
                            rs_rdma(i, j, b).wait_send()
                            ag_rdma(i, j, b).wait_send()
                if i == 0:
                    for jj in (1, 3):
                        for t in (0, 1):
                            slice_rdma(jj, t).wait_send()
                    relay_in_rdma(0).wait_send()
                    relay_in_rdma(1).wait_send()
                if i in (1, 3):
                    relay_fwd_rdma(0 if i == 1 else 1).wait_send()

    return pl.pallas_call(
        body,
        out_shape=jax.ShapeDtypeStruct((B, SQ, D_MODEL), jnp.float32),
        in_specs=[
            pl.BlockSpec(memory_space=pltpu.VMEM),
            pl.BlockSpec(memory_space=pltpu.VMEM),
            pl.BlockSpec(memory_space=pl.ANY),
            pl.BlockSpec(memory_space=pl.ANY),
            pl.BlockSpec(memory_space=pltpu.VMEM),
        ],
        out_specs=pl.BlockSpec(memory_space=pltpu.VMEM),
        scratch_shapes=[
            pltpu.VMEM((2, B, SKV, HQ * DH), jnp.bfloat16),
            pltpu.VMEM((N_DEV - 1, 2, B, SKV, HQ * DH), jnp.bfloat16),
            pltpu.VMEM((B, SKV, HQ * DH), jnp.bfloat16),
            pltpu.VMEM((B, SQ, D_MODEL), jnp.bfloat16),
            pltpu.VMEM((N_DEV, B, QROW, D_MODEL), jnp.bfloat16),
            pltpu.VMEM((B, QROW, D_MODEL), jnp.bfloat16),
            pltpu.VMEM((N_DEV, B, QROW, D_MODEL), jnp.bfloat16),
            pltpu.SemaphoreType.DMA((8,)),
            pltpu.SemaphoreType.DMA((8,)),
            pltpu.SemaphoreType.DMA((2,)),
            pltpu.SemaphoreType.DMA,
            pltpu.SemaphoreType.DMA,
            pltpu.SemaphoreType.DMA((B, N_DEV)),
            pltpu.SemaphoreType.DMA((B, N_DEV)),
            pltpu.SemaphoreType.DMA((B, N_DEV)),
            pltpu.SemaphoreType.DMA((B, N_DEV)),
        ],
        compiler_params=pltpu.CompilerParams(collective_id=0),
    )(xb, wqb, Kb, Vb, wob)
